# baseline (speedup 1.0000x reference)
import functools

import jax
import jax.numpy as jnp
from jax import lax
from jax.experimental import pallas as pl
from jax.experimental.pallas import tpu as pltpu
from jax.experimental.pallas import tpu_sc as plsc

BATCH = 16384
NUM_Q = 1000
_NC = 2
_NS = 16
_NW = _NC * _NS
_BPW = BATCH // _NW
_L = 16
_H = _BPW // 2


def _sc_gather(student_ids, question_ids, ability, difficulty):
    mesh = plsc.VectorSubcoreMesh(core_axis_name="c", subcore_axis_name="s",
                                  num_cores=_NC)

    @functools.partial(
        pl.kernel,
        out_type=(
            jax.ShapeDtypeStruct((BATCH,), jnp.float32),
            jax.ShapeDtypeStruct((BATCH,), jnp.float32),
        ),
        mesh=mesh,
        compiler_params=pltpu.CompilerParams(
            needs_layout_passes=False,
            skip_device_barrier=True,
            disable_bounds_checks=True,
            disable_semaphore_checks=True,
        ),
        scratch_types=[
            pltpu.VMEM((_BPW,), jnp.int32),
            pltpu.VMEM((_BPW,), jnp.int32),
            pltpu.VMEM((_BPW,), jnp.float32),
            pltpu.VMEM((_BPW,), jnp.float32),
            pltpu.VMEM((NUM_Q,), jnp.float32),
            pltpu.SemaphoreType.DMA,
            pltpu.SemaphoreType.DMA,
            pltpu.SemaphoreType.DMA,
            pltpu.SemaphoreType.DMA,
            pltpu.SemaphoreType.DMA,
            pltpu.SemaphoreType.DMA,
        ],
    )
    def gather_kernel(sid_hbm, qid_hbm, abil_hbm, diff_hbm, a_out, d_out,
                      sidx_v, qidx_v, a_v, d_v, dtab_v,
                      sem_d, sem_t, s0, s1, s2, s3):
        wid = lax.axis_index("s") * _NC + lax.axis_index("c")
        base = wid * _BPW
        sems = (s0, s1, s2, s3)
        _Q = _BPW // 4
        lis = [pltpu.async_copy(sid_hbm.at[pl.ds(base + k * _Q, _Q)],
                                sidx_v.at[pl.ds(k * _Q, _Q)], sems[k])
               for k in range(4)]
        lq = pltpu.async_copy(qid_hbm.at[pl.ds(base, _BPW)], qidx_v, sem_d)
        lt = pltpu.async_copy(diff_hbm, dtab_v, sem_t)
        cas = []
        for k in range(4):
            lis[k].wait()
            cas.append(pltpu.async_copy(
                abil_hbm.at[sidx_v.at[pl.ds(k * _Q, _Q)]],
                a_v.at[pl.ds(k * _Q, _Q)], sems[k]))
        lq.wait()
        lt.wait()

        def body(i, carry):
            idx = qidx_v[pl.ds(i * _L, _L)]
            d_v[pl.ds(i * _L, _L)] = plsc.load_gather(dtab_v, [idx])
            return carry

        lax.fori_loop(0, _BPW // _L, body, 0, unroll=1)

        sd = pltpu.async_copy(d_v, d_out.at[pl.ds(base, _BPW)], sem_d)
        sas = []
        for k in range(4):
            cas[k].wait()
            sas.append(pltpu.async_copy(
                a_v.at[pl.ds(k * _Q, _Q)],
                a_out.at[pl.ds(base + k * _Q, _Q)], sems[k]))
        sd.wait()
        for k in range(4):
            sas[k].wait()

    return gather_kernel(student_ids, question_ids, ability, difficulty)


def _tc_body(a_ref, d_ref, lab_ref, pred_ref, loss_ref):
    a = a_ref[...]
    d = d_ref[...]
    lab = lab_ref[...]
    sp_a = jnp.maximum(a, 0.0) + jnp.log1p(jnp.exp(-jnp.abs(a)))
    sp_d = jnp.maximum(d, 0.0) + jnp.log1p(jnp.exp(-jnp.abs(d)))
    pred = sp_a - sp_d
    pred_ref[...] = pred
    t = jnp.maximum(pred, 0.0) - pred * lab + jnp.log1p(jnp.exp(-jnp.abs(pred)))
    loss_ref[...] = (jnp.sum(t) * (1.0 / BATCH)).reshape(1, 1)


def _tc_math(a_raw, d_raw, labels):
    a2 = a_raw.reshape(128, 128)
    d2 = d_raw.reshape(128, 128)
    l2 = labels.reshape(128, 128)
    pred, loss = pl.pallas_call(
        _tc_body,
        out_shape=(
            jax.ShapeDtypeStruct((128, 128), jnp.float32),
            jax.ShapeDtypeStruct((1, 1), jnp.float32),
        ),
    )(a2, d2, l2)
    return loss[0, 0], pred.reshape(BATCH)


def kernel(student_ids, question_ids_collapsed, labels, ability, difficulty):
    a_raw, d_raw = _sc_gather(student_ids, question_ids_collapsed,
                              ability, difficulty)
    return _tc_math(a_raw, d_raw, labels)


# speedup vs baseline: 1.0077x; 1.0077x over previous
import functools

import jax
import jax.numpy as jnp
from jax import lax
from jax.experimental import pallas as pl
from jax.experimental.pallas import tpu as pltpu
from jax.experimental.pallas import tpu_sc as plsc

BATCH = 16384
NUM_Q = 1000
_NC = 2
_NS = 16
_NW = _NC * _NS
_BPW = BATCH // _NW
_L = 16
_H = _BPW // 2


def _sc_gather(student_ids, question_ids, ability, difficulty):
    mesh = plsc.VectorSubcoreMesh(core_axis_name="c", subcore_axis_name="s",
                                  num_cores=_NC)

    @functools.partial(
        pl.kernel,
        out_type=(
            jax.ShapeDtypeStruct((BATCH,), jnp.float32),
            jax.ShapeDtypeStruct((BATCH,), jnp.float32),
        ),
        mesh=mesh,
        compiler_params=pltpu.CompilerParams(
            needs_layout_passes=False,
            skip_device_barrier=True,
            disable_bounds_checks=True,
            disable_semaphore_checks=True,
        ),
        scratch_types=[
            pltpu.VMEM((_BPW,), jnp.int32),
            pltpu.VMEM((_BPW,), jnp.int32),
            pltpu.VMEM((_BPW,), jnp.float32),
            pltpu.VMEM((_BPW,), jnp.float32),
            pltpu.VMEM((NUM_Q,), jnp.float32),
            pltpu.SemaphoreType.DMA,
            pltpu.SemaphoreType.DMA,
            pltpu.SemaphoreType.DMA,
        ],
    )
    def gather_kernel(sid_hbm, qid_hbm, abil_hbm, diff_hbm, a_out, d_out,
                      sidx_v, qidx_v, a_v, d_v, dtab_v, sem_a, sem_d, sem_t):
        wid = lax.axis_index("s") * _NC + lax.axis_index("c")
        base = wid * _BPW
        li1 = pltpu.async_copy(sid_hbm.at[pl.ds(base, _H)],
                               sidx_v.at[pl.ds(0, _H)], sem_a)
        li2 = pltpu.async_copy(sid_hbm.at[pl.ds(base + _H, _H)],
                               sidx_v.at[pl.ds(_H, _H)], sem_t)
        lq = pltpu.async_copy(qid_hbm.at[pl.ds(base, _BPW)], qidx_v, sem_d)
        lt = pltpu.async_copy(diff_hbm, dtab_v, sem_d)
        li1.wait()
        ca = pltpu.async_copy(abil_hbm.at[sidx_v.at[pl.ds(0, _H)]],
                              a_v.at[pl.ds(0, _H)], sem_a)
        li2.wait()
        cb = pltpu.async_copy(abil_hbm.at[sidx_v.at[pl.ds(_H, _H)]],
                              a_v.at[pl.ds(_H, _H)], sem_t)
        lq.wait()
        lt.wait()

        def body(i, carry):
            idx = qidx_v[pl.ds(i * _L, _L)]
            d_v[pl.ds(i * _L, _L)] = plsc.load_gather(dtab_v, [idx])
            return carry

        lax.fori_loop(0, _BPW // _L, body, 0, unroll=1)

        sd = pltpu.async_copy(d_v, d_out.at[pl.ds(base, _BPW)], sem_d)
        ca.wait()
        sa = pltpu.async_copy(a_v.at[pl.ds(0, _H)],
                              a_out.at[pl.ds(base, _H)], sem_a)
        cb.wait()
        sb = pltpu.async_copy(a_v.at[pl.ds(_H, _H)],
                              a_out.at[pl.ds(base + _H, _H)], sem_t)
        sd.wait()
        sa.wait()
        sb.wait()

    return gather_kernel(student_ids, question_ids, ability, difficulty)


def _tc_body(a_ref, d_ref, lab_ref, pred_ref, loss_ref):
    a = a_ref[...]
    d = d_ref[...]
    lab = lab_ref[...]
    sp_a = jnp.maximum(a, 0.0) + jnp.log1p(jnp.exp(-jnp.abs(a)))
    sp_d = jnp.maximum(d, 0.0) + jnp.log1p(jnp.exp(-jnp.abs(d)))
    pred = sp_a - sp_d
    pred_ref[...] = pred
    t = jnp.maximum(pred, 0.0) - pred * lab + jnp.log1p(jnp.exp(-jnp.abs(pred)))
    loss_ref[...] = (jnp.sum(t) * (1.0 / BATCH)).reshape(1, 1)


def _tc_math(a_raw, d_raw, labels):
    a2 = a_raw.reshape(128, 128)
    d2 = d_raw.reshape(128, 128)
    l2 = labels.reshape(128, 128)
    pred, loss = pl.pallas_call(
        _tc_body,
        out_shape=(
            jax.ShapeDtypeStruct((128, 128), jnp.float32),
            jax.ShapeDtypeStruct((1, 1), jnp.float32),
        ),
    )(a2, d2, l2)
    return loss[0, 0], pred.reshape(BATCH)


def kernel(student_ids, question_ids_collapsed, labels, ability, difficulty):
    a_raw, d_raw = _sc_gather(student_ids, question_ids_collapsed,
                              ability, difficulty)
    return _tc_math(a_raw, d_raw, labels)
